# Initial kernel scaffold; baseline (speedup 1.0000x reference)
import jax
import jax.numpy as jnp
from jax.experimental import pallas as pl
from jax.experimental.pallas import tpu as pltpu

H = 128
NH = 8
HD = H // NH
NT = 1024
SCALE = HD ** (-0.5)
BC = 2000
NB = 50


def _ln(x, g, b):
    mu = jnp.mean(x, axis=-1, keepdims=True)
    var = jnp.mean((x - mu) ** 2, axis=-1, keepdims=True)
    return (x - mu) * jax.lax.rsqrt(var + 1e-5) * g + b


def _head_expand():
    r = jax.lax.broadcasted_iota(jnp.int32, (NH, H), 0)
    c = jax.lax.broadcasted_iota(jnp.int32, (NH, H), 1)
    return (r == c // HD).astype(jnp.float32)


def _q_kernel(tis_ref, wqT_ref, bq_ref, q_ref):
    q_ref[...] = (
        jnp.dot(tis_ref[...], wqT_ref[...], preferred_element_type=jnp.float32)
        + bq_ref[...]
    )


def _score_kernel(cell_ref, lab_ref, q_ref, wkT_ref, bk_ref,
                  s_ref, m_ref, m_scr):
    i = pl.program_id(0)

    @pl.when(i == 0)
    def _():
        m_scr[0, 0] = -jnp.inf

    K = (jnp.dot(cell_ref[...], wkT_ref[...],
                 preferred_element_type=jnp.float32) + bk_ref[...])
    lab = lab_ref[0]
    tid = jax.lax.broadcasted_iota(jnp.int32, (1, NT), 1)
    oh = (lab == tid).astype(jnp.bfloat16)
    qg = jnp.dot(oh, q_ref[...].astype(jnp.bfloat16),
                 preferred_element_type=jnp.float32)
    s = jnp.dot(qg * K, _head_expand().T,
                preferred_element_type=jnp.float32) * SCALE
    s_ref[...] = s
    m_scr[0, 0] = jnp.maximum(m_scr[0, 0], jnp.max(s))

    @pl.when(i == NB - 1)
    def _():
        m_ref[0, 0] = m_scr[0, 0]


def _accum_kernel(cell_ref, lab_ref, s_ref, m_ref, wvT_ref, bv_ref,
                  att_ref, dn_ref, att_scr, dn_scr):
    i = pl.program_id(0)

    @pl.when(i == 0)
    def _():
        att_scr[...] = jnp.zeros_like(att_scr)
        dn_scr[...] = jnp.zeros_like(dn_scr)

    V = (jnp.dot(cell_ref[...], wvT_ref[...],
                 preferred_element_type=jnp.float32) + bv_ref[...])
    ex = jnp.exp(s_ref[...] - m_ref[0, 0])
    exR = jnp.dot(ex, _head_expand(),
                  preferred_element_type=jnp.float32)
    exV = (V * exR).astype(jnp.bfloat16)
    lab = lab_ref[0]
    tid = jax.lax.broadcasted_iota(jnp.int32, (NT, 1), 0)
    ohT = (tid == lab).astype(jnp.bfloat16)
    att_scr[...] += jnp.dot(ohT, exV, preferred_element_type=jnp.float32)
    sel = (jax.lax.broadcasted_iota(jnp.int32, (NH, 16), 0)
           == jax.lax.broadcasted_iota(jnp.int32, (NH, 16), 1)
           ).astype(jnp.float32)
    col8 = (jax.lax.broadcasted_iota(jnp.int32, (BC, 16), 1) == 8
            ).astype(jnp.float32)
    dn16 = (jnp.dot(ex, sel, preferred_element_type=jnp.float32)
            + col8).astype(jnp.bfloat16)
    dn_scr[...] += jnp.dot(ohT, dn16, preferred_element_type=jnp.float32)

    @pl.when(i == NB - 1)
    def _():
        att_ref[...] = att_scr[...]
        dn_ref[...] = dn_scr[...]


def _tissue_kernel(att_ref, dn_ref, tis_ref, woT_ref, bo_ref,
                   tdwvT_ref, tdbv_ref, tdwoT_ref, tdbo_ref,
                   g_ref, b_ref, td_ref, tout_ref):
    dn = dn_ref[...]
    counts = dn[:, 8:9]
    mask = counts > 0.5
    denom = dn[:, :NH]
    denom = jnp.where(denom == 0.0, 1.0, denom)
    rep = jnp.dot(1.0 / denom, _head_expand(),
                  preferred_element_type=jnp.float32)
    att = att_ref[...] * rep
    att_o = (jnp.dot(att, woT_ref[...],
                     preferred_element_type=jnp.float32) + bo_ref[...])
    tis = tis_ref[...]
    t_upd = jnp.where(mask, att_o, tis)
    td_v = (jnp.dot(t_upd, tdwvT_ref[...],
                    preferred_element_type=jnp.float32) + tdbv_ref[...])
    td_ref[...] = (jnp.dot(td_v, tdwoT_ref[...],
                           preferred_element_type=jnp.float32) + tdbo_ref[...])
    tout_ref[...] = _ln(tis + t_upd, g_ref[...], b_ref[...])


def _cell_kernel(cell_ref, lab_ref, td_ref, g_ref, b_ref, out_ref):
    lab = lab_ref[0]
    tid = jax.lax.broadcasted_iota(jnp.int32, (1, NT), 1)
    oh = (lab == tid).astype(jnp.bfloat16)
    td = td_ref[...]
    hi = td.astype(jnp.bfloat16)
    lo = (td - hi.astype(jnp.float32)).astype(jnp.bfloat16)
    G = (jnp.dot(oh, hi, preferred_element_type=jnp.float32)
         + jnp.dot(oh, lo, preferred_element_type=jnp.float32))
    out_ref[...] = _ln(cell_ref[...] + G, g_ref[...], b_ref[...])


def _full(shape):
    return pl.BlockSpec(shape, lambda i: tuple(0 for _ in shape))


def kernel(cell_features, tissue_features, cluster_labels, tissue_batch,
           bu_Wq, bu_bq, bu_Wk, bu_bk, bu_Wv, bu_bv, bu_Wo, bu_bo,
           td_Wq, td_bq, td_Wk, td_bk, td_Wv, td_bv, td_Wo, td_bo,
           cell_ln_g, cell_ln_b, tissue_ln_g, tissue_ln_b):
    NC = cell_features.shape[0]
    lab_col = cluster_labels.reshape(NB, BC, 1)
    lab_row = cluster_labels.reshape(NB, 1, BC)

    Q = pl.pallas_call(
        _q_kernel,
        out_shape=jax.ShapeDtypeStruct((NT, H), jnp.float32),
    )(tissue_features, bu_Wq.T, bu_bq.reshape(1, H))

    scores, M = pl.pallas_call(
        _score_kernel,
        grid=(NB,),
        in_specs=[
            pl.BlockSpec((BC, H), lambda i: (i, 0)),
            pl.BlockSpec((1, BC, 1), lambda i: (i, 0, 0)),
            _full((NT, H)),
            _full((H, H)),
            _full((1, H)),
        ],
        out_specs=[
            pl.BlockSpec((BC, NH), lambda i: (i, 0)),
            pl.BlockSpec((1, 1), lambda i: (0, 0)),
        ],
        out_shape=[
            jax.ShapeDtypeStruct((NC, NH), jnp.float32),
            jax.ShapeDtypeStruct((1, 1), jnp.float32),
        ],
        scratch_shapes=[pltpu.SMEM((1, 1), jnp.float32)],
    )(cell_features, lab_col, Q, bu_Wk.T, bu_bk.reshape(1, H))

    att_raw, dn = pl.pallas_call(
        _accum_kernel,
        grid=(NB,),
        in_specs=[
            pl.BlockSpec((BC, H), lambda i: (i, 0)),
            pl.BlockSpec((1, 1, BC), lambda i: (i, 0, 0)),
            pl.BlockSpec((BC, NH), lambda i: (i, 0)),
            _full((1, 1)),
            _full((H, H)),
            _full((1, H)),
        ],
        out_specs=[
            pl.BlockSpec((NT, H), lambda i: (0, 0)),
            pl.BlockSpec((NT, 16), lambda i: (0, 0)),
        ],
        out_shape=[
            jax.ShapeDtypeStruct((NT, H), jnp.float32),
            jax.ShapeDtypeStruct((NT, 16), jnp.float32),
        ],
        scratch_shapes=[
            pltpu.VMEM((NT, H), jnp.float32),
            pltpu.VMEM((NT, 16), jnp.float32),
        ],
    )(cell_features, lab_row, scores, M, bu_Wv.T, bu_bv.reshape(1, H))

    td_out, tissue_out = pl.pallas_call(
        _tissue_kernel,
        out_shape=[
            jax.ShapeDtypeStruct((NT, H), jnp.float32),
            jax.ShapeDtypeStruct((NT, H), jnp.float32),
        ],
    )(att_raw, dn, tissue_features, bu_Wo.T, bu_bo.reshape(1, H),
      td_Wv.T, td_bv.reshape(1, H), td_Wo.T, td_bo.reshape(1, H),
      tissue_ln_g.reshape(1, H), tissue_ln_b.reshape(1, H))

    cell_out = pl.pallas_call(
        _cell_kernel,
        grid=(NB,),
        in_specs=[
            pl.BlockSpec((BC, H), lambda i: (i, 0)),
            pl.BlockSpec((1, BC, 1), lambda i: (i, 0, 0)),
            _full((NT, H)),
            _full((1, H)),
            _full((1, H)),
        ],
        out_specs=pl.BlockSpec((BC, H), lambda i: (i, 0)),
        out_shape=jax.ShapeDtypeStruct((NC, H), jnp.float32),
    )(cell_features, lab_col, td_out,
      cell_ln_g.reshape(1, H), cell_ln_b.reshape(1, H))

    return cell_out, tissue_out


# speedup vs baseline: 20.7422x; 20.7422x over previous
import jax
import jax.numpy as jnp
from jax.experimental import pallas as pl
from jax.experimental.pallas import tpu as pltpu

H = 128
NH = 8
HD = H // NH
NT = 1024
SCALE = HD ** (-0.5)
BC = 2000
NB = 50


def _ln(x, g, b):
    mu = jnp.mean(x, axis=-1, keepdims=True)
    var = jnp.mean((x - mu) ** 2, axis=-1, keepdims=True)
    return (x - mu) * jax.lax.rsqrt(var + 1e-5) * g + b


def _head_expand():
    r = jax.lax.broadcasted_iota(jnp.int32, (NH, H), 0)
    c = jax.lax.broadcasted_iota(jnp.int32, (NH, H), 1)
    return (r == c // HD).astype(jnp.float32)


def _q_kernel(tis_ref, wqT_ref, bq_ref, q_ref):
    q_ref[...] = (
        jnp.dot(tis_ref[...], wqT_ref[...], preferred_element_type=jnp.float32)
        + bq_ref[...]
    )


def _score_kernel(cell_ref, lab_ref, q_ref, wkT_ref, bk_ref,
                  s_ref, m_ref, m_scr):
    i = pl.program_id(0)

    @pl.when(i == 0)
    def _():
        m_scr[0, 0] = -jnp.inf

    K = (jnp.dot(cell_ref[...], wkT_ref[...],
                 preferred_element_type=jnp.float32) + bk_ref[...])
    lab = lab_ref[0]
    tid = jax.lax.broadcasted_iota(jnp.int32, (1, NT), 1)
    oh = (lab == tid).astype(jnp.bfloat16)
    qg = jnp.dot(oh, q_ref[...].astype(jnp.bfloat16),
                 preferred_element_type=jnp.float32)
    s = jnp.dot(qg * K, _head_expand().T,
                preferred_element_type=jnp.float32) * SCALE
    s_ref[...] = s
    m_scr[0, 0] = jnp.maximum(m_scr[0, 0], jnp.max(s))

    @pl.when(i == NB - 1)
    def _():
        m_ref[...] = jnp.full((1, 1), m_scr[0, 0], jnp.float32)


def _accum_kernel(cell_ref, lab_ref, s_ref, m_ref, wvT_ref, bv_ref,
                  att_ref, dn_ref, att_scr, dn_scr):
    i = pl.program_id(0)

    @pl.when(i == 0)
    def _():
        att_scr[...] = jnp.zeros_like(att_scr)
        dn_scr[...] = jnp.zeros_like(dn_scr)

    V = (jnp.dot(cell_ref[...], wvT_ref[...],
                 preferred_element_type=jnp.float32) + bv_ref[...])
    ex = jnp.exp(s_ref[...] - m_ref[...])
    exR = jnp.dot(ex, _head_expand(),
                  preferred_element_type=jnp.float32)
    exV = (V * exR).astype(jnp.bfloat16)
    lab = lab_ref[0]
    tid = jax.lax.broadcasted_iota(jnp.int32, (NT, 1), 0)
    ohT = (tid == lab).astype(jnp.bfloat16)
    att_scr[...] += jnp.dot(ohT, exV, preferred_element_type=jnp.float32)
    sel = (jax.lax.broadcasted_iota(jnp.int32, (NH, 16), 0)
           == jax.lax.broadcasted_iota(jnp.int32, (NH, 16), 1)
           ).astype(jnp.float32)
    col8 = (jax.lax.broadcasted_iota(jnp.int32, (BC, 16), 1) == 8
            ).astype(jnp.float32)
    dn16 = (jnp.dot(ex, sel, preferred_element_type=jnp.float32)
            + col8).astype(jnp.bfloat16)
    dn_scr[...] += jnp.dot(ohT, dn16, preferred_element_type=jnp.float32)

    @pl.when(i == NB - 1)
    def _():
        att_ref[...] = att_scr[...]
        dn_ref[...] = dn_scr[...]


def _tissue_kernel(att_ref, dn_ref, tis_ref, woT_ref, bo_ref,
                   tdwvT_ref, tdbv_ref, tdwoT_ref, tdbo_ref,
                   g_ref, b_ref, td_ref, tout_ref):
    dn = dn_ref[...]
    counts = dn[:, 8:9]
    mask = counts > 0.5
    denom = dn[:, :NH]
    denom = jnp.where(denom == 0.0, 1.0, denom)
    rep = jnp.dot(1.0 / denom, _head_expand(),
                  preferred_element_type=jnp.float32)
    att = att_ref[...] * rep
    att_o = (jnp.dot(att, woT_ref[...],
                     preferred_element_type=jnp.float32) + bo_ref[...])
    tis = tis_ref[...]
    t_upd = jnp.where(mask, att_o, tis)
    td_v = (jnp.dot(t_upd, tdwvT_ref[...],
                    preferred_element_type=jnp.float32) + tdbv_ref[...])
    td_ref[...] = (jnp.dot(td_v, tdwoT_ref[...],
                           preferred_element_type=jnp.float32) + tdbo_ref[...])
    tout_ref[...] = _ln(tis + t_upd, g_ref[...], b_ref[...])


def _cell_kernel(cell_ref, lab_ref, td_ref, g_ref, b_ref, out_ref):
    lab = lab_ref[0]
    tid = jax.lax.broadcasted_iota(jnp.int32, (1, NT), 1)
    oh = (lab == tid).astype(jnp.bfloat16)
    td = td_ref[...]
    hi = td.astype(jnp.bfloat16)
    lo = (td - hi.astype(jnp.float32)).astype(jnp.bfloat16)
    G = (jnp.dot(oh, hi, preferred_element_type=jnp.float32)
         + jnp.dot(oh, lo, preferred_element_type=jnp.float32))
    out_ref[...] = _ln(cell_ref[...] + G, g_ref[...], b_ref[...])


def _full(shape):
    return pl.BlockSpec(shape, lambda i: tuple(0 for _ in shape))


def kernel(cell_features, tissue_features, cluster_labels, tissue_batch,
           bu_Wq, bu_bq, bu_Wk, bu_bk, bu_Wv, bu_bv, bu_Wo, bu_bo,
           td_Wq, td_bq, td_Wk, td_bk, td_Wv, td_bv, td_Wo, td_bo,
           cell_ln_g, cell_ln_b, tissue_ln_g, tissue_ln_b):
    NC = cell_features.shape[0]
    lab_col = cluster_labels.reshape(NB, BC, 1)
    lab_row = cluster_labels.reshape(NB, 1, BC)

    Q = pl.pallas_call(
        _q_kernel,
        out_shape=jax.ShapeDtypeStruct((NT, H), jnp.float32),
    )(tissue_features, bu_Wq.T, bu_bq.reshape(1, H))

    scores, M = pl.pallas_call(
        _score_kernel,
        grid=(NB,),
        in_specs=[
            pl.BlockSpec((BC, H), lambda i: (i, 0)),
            pl.BlockSpec((1, BC, 1), lambda i: (i, 0, 0)),
            _full((NT, H)),
            _full((H, H)),
            _full((1, H)),
        ],
        out_specs=[
            pl.BlockSpec((BC, NH), lambda i: (i, 0)),
            pl.BlockSpec((1, 1), lambda i: (0, 0)),
        ],
        out_shape=[
            jax.ShapeDtypeStruct((NC, NH), jnp.float32),
            jax.ShapeDtypeStruct((1, 1), jnp.float32),
        ],
        scratch_shapes=[pltpu.SMEM((1, 1), jnp.float32)],
    )(cell_features, lab_col, Q, bu_Wk.T, bu_bk.reshape(1, H))

    att_raw, dn = pl.pallas_call(
        _accum_kernel,
        grid=(NB,),
        in_specs=[
            pl.BlockSpec((BC, H), lambda i: (i, 0)),
            pl.BlockSpec((1, 1, BC), lambda i: (i, 0, 0)),
            pl.BlockSpec((BC, NH), lambda i: (i, 0)),
            _full((1, 1)),
            _full((H, H)),
            _full((1, H)),
        ],
        out_specs=[
            pl.BlockSpec((NT, H), lambda i: (0, 0)),
            pl.BlockSpec((NT, 16), lambda i: (0, 0)),
        ],
        out_shape=[
            jax.ShapeDtypeStruct((NT, H), jnp.float32),
            jax.ShapeDtypeStruct((NT, 16), jnp.float32),
        ],
        scratch_shapes=[
            pltpu.VMEM((NT, H), jnp.float32),
            pltpu.VMEM((NT, 16), jnp.float32),
        ],
    )(cell_features, lab_row, scores, M, bu_Wv.T, bu_bv.reshape(1, H))

    td_out, tissue_out = pl.pallas_call(
        _tissue_kernel,
        out_shape=[
            jax.ShapeDtypeStruct((NT, H), jnp.float32),
            jax.ShapeDtypeStruct((NT, H), jnp.float32),
        ],
    )(att_raw, dn, tissue_features, bu_Wo.T, bu_bo.reshape(1, H),
      td_Wv.T, td_bv.reshape(1, H), td_Wo.T, td_bo.reshape(1, H),
      tissue_ln_g.reshape(1, H), tissue_ln_b.reshape(1, H))

    cell_out = pl.pallas_call(
        _cell_kernel,
        grid=(NB,),
        in_specs=[
            pl.BlockSpec((BC, H), lambda i: (i, 0)),
            pl.BlockSpec((1, BC, 1), lambda i: (i, 0, 0)),
            _full((NT, H)),
            _full((1, H)),
            _full((1, H)),
        ],
        out_specs=pl.BlockSpec((BC, H), lambda i: (i, 0)),
        out_shape=jax.ShapeDtypeStruct((NC, H), jnp.float32),
    )(cell_features, lab_col, td_out,
      cell_ln_g.reshape(1, H), cell_ln_b.reshape(1, H))

    return cell_out, tissue_out
